# speedup vs baseline: 31.0932x; 31.0932x over previous
import functools

import jax
import jax.numpy as jnp
from jax.experimental import pallas as pl

N = 4096
D = N + 1
R = 512
G = N // R


def _coeffs(l, u):
    safe = lambda x: jnp.where(x == 0, jnp.ones_like(x), x)
    lam = u / safe(u - l)
    alpha_c = jnp.where(u < -l, 1e-5, 1.0)
    au_h = jnp.where(u - 6.0 < 6.0 - l, 6.0 / safe(6.0 - l), 1e-5)
    al_h = jnp.where(u < -l, 1e-5, 6.0 / safe(u))
    lam_m = (6.0 - l) / safe(u - l)
    alpha_m = jnp.where(u - 6.0 < 6.0 - l, 1.0, 1e-5)
    m_pos = (u > 0) & (u <= 6) & (l >= 0)
    m_cross = (u > 0) & (u <= 6) & (l < 0)
    m_hcross = (u > 6) & (l <= 0)
    m_mid = (u > 6) & (l > 0) & (l <= 6)
    m_sat = (u > 6) & (l > 6)
    diag_low = jnp.where(m_pos, 1.0, jnp.where(m_cross, alpha_c, jnp.where(m_hcross, al_h, jnp.where(m_mid, lam_m, 0.0))))
    bias_low = jnp.where(m_mid, l * (1.0 - lam_m), jnp.where(m_sat, 6.0, 0.0))
    diag_up = jnp.where(m_pos, 1.0, jnp.where(m_cross, lam, jnp.where(m_hcross, au_h, jnp.where(m_mid, alpha_m, 0.0))))
    bias_up = jnp.where(m_cross, -lam * l, jnp.where(m_hcross, 6.0 * (1.0 - au_h), jnp.where(m_mid, 6.0 * (1.0 - alpha_m), jnp.where(m_sat, 6.0, 0.0))))
    clb = jnp.where(m_pos, l, jnp.where(m_cross, alpha_c * l, jnp.where(m_hcross, al_h * l, jnp.where(m_mid, l, jnp.where(m_sat, 6.0, 0.0)))))
    cub = jnp.where(m_pos, u, jnp.where(m_cross, u, jnp.where(m_hcross, 6.0 + au_h * (u - 6.0), jnp.where(m_mid | m_sat, 6.0, 0.0))))
    return diag_low, bias_low, diag_up, bias_up, clb, cub


def _coeff_kernel(l_ref, u_ref, dl_ref, du_ref, bl_ref, bu_ref, clb_ref, cub_ref):
    l = l_ref[...]
    u = u_ref[...]
    diag_low, bias_low, diag_up, bias_up, clb, cub = _coeffs(l, u)
    dl_ref[...] = diag_low
    du_ref[...] = diag_up
    bl_ref[:, :N] = bias_low
    bl_ref[:, N:] = jnp.ones((1, 1), jnp.float32)
    bu_ref[:, :N] = bias_up
    bu_ref[:, N:] = jnp.ones((1, 1), jnp.float32)
    clb_ref[...] = clb
    cub_ref[...] = cub


def _matrix_kernel(dl_ref, du_ref, bl_ref, bu_ref, alb_ref, aub_ref):
    i = pl.program_id(0)

    @pl.when(i < G)
    def _main():
        alb_ref[...] = jnp.zeros((R, D), jnp.float32)
        aub_ref[...] = jnp.zeros((R, D), jnp.float32)
        r0 = jax.lax.broadcasted_iota(jnp.int32, (R, R), 0)
        r1 = jax.lax.broadcasted_iota(jnp.int32, (R, R), 1)
        on_diag = r0 == r1
        alb_ref[:, pl.ds(i * R, R)] = jnp.where(on_diag, dl_ref[...], 0.0)
        aub_ref[:, pl.ds(i * R, R)] = jnp.where(on_diag, du_ref[...], 0.0)

    @pl.when(i == G)
    def _bias_row():
        alb_ref[...] = jnp.broadcast_to(bl_ref[...], (R, D))
        aub_ref[...] = jnp.broadcast_to(bu_ref[...], (R, D))


@functools.partial(jax.jit, static_argnames=())
def kernel(concrete_lower, concrete_upper, abstract_lower_in, abstract_upper_in):
    l_row = concrete_lower.reshape(1, N)
    u_row = concrete_upper.reshape(1, N)

    dl, du, bl, bu, clb, cub = pl.pallas_call(
        _coeff_kernel,
        out_shape=(
            jax.ShapeDtypeStruct((1, N), jnp.float32),
            jax.ShapeDtypeStruct((1, N), jnp.float32),
            jax.ShapeDtypeStruct((1, D), jnp.float32),
            jax.ShapeDtypeStruct((1, D), jnp.float32),
            jax.ShapeDtypeStruct((1, N), jnp.float32),
            jax.ShapeDtypeStruct((1, N), jnp.float32),
        ),
    )(l_row, u_row)

    dl_col = jnp.pad(dl.reshape(N), (0, R)).reshape(N + R, 1)
    du_col = jnp.pad(du.reshape(N), (0, R)).reshape(N + R, 1)

    alb, aub = pl.pallas_call(
        _matrix_kernel,
        grid=(G + 1,),
        in_specs=[
            pl.BlockSpec((R, 1), lambda i: (i, 0)),
            pl.BlockSpec((R, 1), lambda i: (i, 0)),
            pl.BlockSpec((1, D), lambda i: (0, 0)),
            pl.BlockSpec((1, D), lambda i: (0, 0)),
        ],
        out_specs=(
            pl.BlockSpec((R, D), lambda i: (i, 0)),
            pl.BlockSpec((R, D), lambda i: (i, 0)),
        ),
        out_shape=(
            jax.ShapeDtypeStruct((D, D), jnp.float32),
            jax.ShapeDtypeStruct((D, D), jnp.float32),
        ),
    )(dl_col, du_col, bl, bu)

    return ((clb.reshape(N), cub.reshape(N)), (alb, aub))


# speedup vs baseline: 32.9637x; 1.0602x over previous
import functools

import jax
import jax.numpy as jnp
from jax.experimental import pallas as pl

N = 4096
D = N + 1
R = 256
G = N // R


def _coeffs(l, u):
    safe = lambda x: jnp.where(x == 0, jnp.ones_like(x), x)
    lam = u / safe(u - l)
    alpha_c = jnp.where(u < -l, 1e-5, 1.0)
    au_h = jnp.where(u - 6.0 < 6.0 - l, 6.0 / safe(6.0 - l), 1e-5)
    al_h = jnp.where(u < -l, 1e-5, 6.0 / safe(u))
    lam_m = (6.0 - l) / safe(u - l)
    alpha_m = jnp.where(u - 6.0 < 6.0 - l, 1.0, 1e-5)
    m_pos = (u > 0) & (u <= 6) & (l >= 0)
    m_cross = (u > 0) & (u <= 6) & (l < 0)
    m_hcross = (u > 6) & (l <= 0)
    m_mid = (u > 6) & (l > 0) & (l <= 6)
    m_sat = (u > 6) & (l > 6)
    diag_low = jnp.where(m_pos, 1.0, jnp.where(m_cross, alpha_c, jnp.where(m_hcross, al_h, jnp.where(m_mid, lam_m, 0.0))))
    bias_low = jnp.where(m_mid, l * (1.0 - lam_m), jnp.where(m_sat, 6.0, 0.0))
    diag_up = jnp.where(m_pos, 1.0, jnp.where(m_cross, lam, jnp.where(m_hcross, au_h, jnp.where(m_mid, alpha_m, 0.0))))
    bias_up = jnp.where(m_cross, -lam * l, jnp.where(m_hcross, 6.0 * (1.0 - au_h), jnp.where(m_mid, 6.0 * (1.0 - alpha_m), jnp.where(m_sat, 6.0, 0.0))))
    clb = jnp.where(m_pos, l, jnp.where(m_cross, alpha_c * l, jnp.where(m_hcross, al_h * l, jnp.where(m_mid, l, jnp.where(m_sat, 6.0, 0.0)))))
    cub = jnp.where(m_pos, u, jnp.where(m_cross, u, jnp.where(m_hcross, 6.0 + au_h * (u - 6.0), jnp.where(m_mid | m_sat, 6.0, 0.0))))
    return diag_low, bias_low, diag_up, bias_up, clb, cub


def _coeff_kernel(l_ref, u_ref, dl_ref, du_ref, bl_ref, bu_ref, clb_ref, cub_ref):
    l = l_ref[...]
    u = u_ref[...]
    diag_low, bias_low, diag_up, bias_up, clb, cub = _coeffs(l, u)
    dl_ref[...] = diag_low
    du_ref[...] = diag_up
    bl_ref[:, :N] = bias_low
    bl_ref[:, N:] = jnp.ones((1, 1), jnp.float32)
    bu_ref[:, :N] = bias_up
    bu_ref[:, N:] = jnp.ones((1, 1), jnp.float32)
    clb_ref[...] = clb
    cub_ref[...] = cub


def _matrix_kernel(dl_ref, du_ref, bl_ref, bu_ref, alb_ref, aub_ref):
    i = pl.program_id(0)

    @pl.when(i < G)
    def _main():
        alb_ref[...] = jnp.zeros((R, D), jnp.float32)
        aub_ref[...] = jnp.zeros((R, D), jnp.float32)
        r0 = jax.lax.broadcasted_iota(jnp.int32, (R, R), 0)
        r1 = jax.lax.broadcasted_iota(jnp.int32, (R, R), 1)
        on_diag = r0 == r1
        alb_ref[:, pl.ds(i * R, R)] = jnp.where(on_diag, dl_ref[...], 0.0)
        aub_ref[:, pl.ds(i * R, R)] = jnp.where(on_diag, du_ref[...], 0.0)

    @pl.when(i == G)
    def _bias_row():
        alb_ref[...] = jnp.broadcast_to(bl_ref[...], (R, D))
        aub_ref[...] = jnp.broadcast_to(bu_ref[...], (R, D))


@functools.partial(jax.jit, static_argnames=())
def kernel(concrete_lower, concrete_upper, abstract_lower_in, abstract_upper_in):
    l_row = concrete_lower.reshape(1, N)
    u_row = concrete_upper.reshape(1, N)

    dl, du, bl, bu, clb, cub = pl.pallas_call(
        _coeff_kernel,
        out_shape=(
            jax.ShapeDtypeStruct((1, N), jnp.float32),
            jax.ShapeDtypeStruct((1, N), jnp.float32),
            jax.ShapeDtypeStruct((1, D), jnp.float32),
            jax.ShapeDtypeStruct((1, D), jnp.float32),
            jax.ShapeDtypeStruct((1, N), jnp.float32),
            jax.ShapeDtypeStruct((1, N), jnp.float32),
        ),
    )(l_row, u_row)

    dl_col = jnp.pad(dl.reshape(N), (0, R)).reshape(N + R, 1)
    du_col = jnp.pad(du.reshape(N), (0, R)).reshape(N + R, 1)

    alb, aub = pl.pallas_call(
        _matrix_kernel,
        grid=(G + 1,),
        in_specs=[
            pl.BlockSpec((R, 1), lambda i: (i, 0)),
            pl.BlockSpec((R, 1), lambda i: (i, 0)),
            pl.BlockSpec((1, D), lambda i: (0, 0)),
            pl.BlockSpec((1, D), lambda i: (0, 0)),
        ],
        out_specs=(
            pl.BlockSpec((R, D), lambda i: (i, 0)),
            pl.BlockSpec((R, D), lambda i: (i, 0)),
        ),
        out_shape=(
            jax.ShapeDtypeStruct((D, D), jnp.float32),
            jax.ShapeDtypeStruct((D, D), jnp.float32),
        ),
    )(dl_col, du_col, bl, bu)

    return ((clb.reshape(N), cub.reshape(N)), (alb, aub))
